# initial kernel scaffold (unmeasured)
import jax
import jax.numpy as jnp
from jax import lax
from jax.experimental import pallas as pl
from jax.experimental.pallas import tpu as pltpu

N_DEV = 4
N_LAYERS = 3
B = 128
D = 128


def kernel(x, Win0, Wout0, Win1, Wout1, Win2, Wout2):
    def body(
        x_ref,
        win0_ref,
        wout0_ref,
        win1_ref,
        wout1_ref,
        win2_ref,
        wout2_ref,
        out_ref,
        stage_ref,
        comm_ref,
        send_sems,
        recv_sems,
    ):
        my = lax.axis_index("i")

        def layer(xin, win_ref, wout_ref):
            h = jnp.dot(
                xin.astype(jnp.bfloat16),
                win_ref[:, :].astype(jnp.bfloat16),
                preferred_element_type=jnp.float32,
            )
            h = jnp.maximum(h, 0.0)
            return jnp.dot(
                h.astype(jnp.bfloat16),
                wout_ref[:, :].astype(jnp.bfloat16),
                preferred_element_type=jnp.float32,
            )

        xin = x_ref[:, :]
        wpairs = [(win0_ref, wout0_ref), (win1_ref, wout1_ref), (win2_ref, wout2_ref)]
        for r, (win_ref, wout_ref) in enumerate(wpairs):
            partial = layer(xin, win_ref, wout_ref)
            stage_ref[r] = partial
            rdmas = []
            for k in range(1, N_DEV):
                rdma = pltpu.make_async_remote_copy(
                    src_ref=stage_ref.at[r],
                    dst_ref=comm_ref.at[r, k - 1],
                    send_sem=send_sems.at[r, k - 1],
                    recv_sem=recv_sems.at[r, k - 1],
                    device_id=((my + k) % N_DEV,),
                    device_id_type=pl.DeviceIdType.MESH,
                )
                rdma.start()
                rdmas.append(rdma)
            for rdma in rdmas:
                rdma.wait()
            xin = partial + comm_ref[r, 0] + comm_ref[r, 1] + comm_ref[r, 2]

        out_ref[:, :] = lax.dynamic_slice_in_dim(xin, my * (B // N_DEV), B // N_DEV)

    return pl.pallas_call(
        body,
        out_shape=jax.ShapeDtypeStruct((B // N_DEV, D), jnp.float32),
        in_specs=[pl.BlockSpec(memory_space=pltpu.VMEM)] * 7,
        out_specs=pl.BlockSpec(memory_space=pltpu.VMEM),
        scratch_shapes=[
            pltpu.VMEM((N_LAYERS, B, D), jnp.float32),
            pltpu.VMEM((N_LAYERS, N_DEV - 1, B, D), jnp.float32),
            pltpu.SemaphoreType.DMA((N_LAYERS, N_DEV - 1)),
            pltpu.SemaphoreType.DMA((N_LAYERS, N_DEV - 1)),
        ],
    )(x, Win0, Wout0, Win1, Wout1, Win2, Wout2)


# baseline (device time: 23805 ns/iter reference)
import jax
import jax.numpy as jnp
from jax import lax
from jax.experimental import pallas as pl
from jax.experimental.pallas import tpu as pltpu

N_DEV = 4
N_LAYERS = 3
B = 128
D = 128


def kernel(x, Win0, Wout0, Win1, Wout1, Win2, Wout2):
    def body(
        x_ref,
        win0_ref,
        wout0_ref,
        win1_ref,
        wout1_ref,
        win2_ref,
        wout2_ref,
        out_ref,
        stage_ref,
        comm_ref,
        total_ref,
        send_sems,
        recv_sems,
    ):
        my = lax.axis_index("i")

        def layer(xin, win_ref, wout_ref):
            h = jnp.dot(
                xin.astype(jnp.bfloat16),
                win_ref[:, :].astype(jnp.bfloat16),
                preferred_element_type=jnp.float32,
            )
            h = jnp.maximum(h, 0.0)
            return jnp.dot(
                h.astype(jnp.bfloat16),
                wout_ref[:, :].astype(jnp.bfloat16),
                preferred_element_type=jnp.float32,
            )

        xin = x_ref[:, :]
        wpairs = [(win0_ref, wout0_ref), (win1_ref, wout1_ref), (win2_ref, wout2_ref)]
        for r, (win_ref, wout_ref) in enumerate(wpairs):
            partial = layer(xin, win_ref, wout_ref)
            stage_ref[r] = partial
            rdmas = []
            for k in range(1, N_DEV):
                rdma = pltpu.make_async_remote_copy(
                    src_ref=stage_ref.at[r],
                    dst_ref=comm_ref.at[r, k - 1],
                    send_sem=send_sems.at[r, k - 1],
                    recv_sem=recv_sems.at[r, k - 1],
                    device_id=((my + k) % N_DEV,),
                    device_id_type=pl.DeviceIdType.MESH,
                )
                rdma.start()
                rdmas.append(rdma)
            for rdma in rdmas:
                rdma.wait()
            xin = partial + comm_ref[r, 0] + comm_ref[r, 1] + comm_ref[r, 2]

        total_ref[:, :] = xin
        out_ref[:, :] = total_ref[pl.ds(my * (B // N_DEV), B // N_DEV), :]

    return pl.pallas_call(
        body,
        out_shape=jax.ShapeDtypeStruct((B // N_DEV, D), jnp.float32),
        in_specs=[pl.BlockSpec(memory_space=pltpu.VMEM)] * 7,
        out_specs=pl.BlockSpec(memory_space=pltpu.VMEM),
        scratch_shapes=[
            pltpu.VMEM((N_LAYERS, B, D), jnp.float32),
            pltpu.VMEM((N_LAYERS, N_DEV - 1, B, D), jnp.float32),
            pltpu.VMEM((B, D), jnp.float32),
            pltpu.SemaphoreType.DMA((N_LAYERS, N_DEV - 1)),
            pltpu.SemaphoreType.DMA((N_LAYERS, N_DEV - 1)),
        ],
    )(x, Win0, Wout0, Win1, Wout1, Win2, Wout2)


# device time: 17864 ns/iter; 1.3326x vs baseline; 1.3326x over previous
import jax
import jax.numpy as jnp
from jax import lax
from jax.experimental import pallas as pl
from jax.experimental.pallas import tpu as pltpu

N_DEV = 4
N_LAYERS = 3
B = 128
D = 128


def kernel(x, Win0, Wout0, Win1, Wout1, Win2, Wout2):
    def body(
        x_ref,
        win0_ref,
        wout0_ref,
        win1_ref,
        wout1_ref,
        win2_ref,
        wout2_ref,
        out_ref,
        stage_ref,
        comm_ref,
        total_ref,
        send_sems,
        recv_sems,
    ):
        my = lax.axis_index("i")

        barrier_sem = pltpu.get_barrier_semaphore()
        for k in range(1, N_DEV):
            pl.semaphore_signal(
                barrier_sem,
                inc=1,
                device_id=((my + k) % N_DEV,),
                device_id_type=pl.DeviceIdType.MESH,
            )
        pl.semaphore_wait(barrier_sem, N_DEV - 1)

        def layer(xin, win_ref, wout_ref):
            h = jnp.dot(
                xin,
                win_ref[:, :].astype(jnp.bfloat16),
                preferred_element_type=jnp.float32,
            )
            h = jnp.maximum(h, 0.0)
            return jnp.dot(
                h.astype(jnp.bfloat16),
                wout_ref[:, :].astype(jnp.bfloat16),
                preferred_element_type=jnp.float32,
            )

        xin = x_ref[:, :].astype(jnp.bfloat16)
        wpairs = [(win0_ref, wout0_ref), (win1_ref, wout1_ref), (win2_ref, wout2_ref)]
        for r, (win_ref, wout_ref) in enumerate(wpairs):
            partial = layer(xin, win_ref, wout_ref)
            stage_ref[r] = partial.astype(jnp.bfloat16)
            rdmas = []
            for k in (2, 1, 3):
                rdma = pltpu.make_async_remote_copy(
                    src_ref=stage_ref.at[r],
                    dst_ref=comm_ref.at[r, k - 1],
                    send_sem=send_sems.at[r, k - 1],
                    recv_sem=recv_sems.at[r, k - 1],
                    device_id=((my + k) % N_DEV,),
                    device_id_type=pl.DeviceIdType.MESH,
                )
                rdma.start()
                rdmas.append(rdma)
            for rdma in rdmas:
                rdma.wait_recv()
            total = (
                partial
                + comm_ref[r, 0].astype(jnp.float32)
                + comm_ref[r, 1].astype(jnp.float32)
                + comm_ref[r, 2].astype(jnp.float32)
            )
            for rdma in rdmas:
                rdma.wait_send()
            if r < N_LAYERS - 1:
                xin = total.astype(jnp.bfloat16)

        total_ref[:, :] = total
        out_ref[:, :] = total_ref[pl.ds(my * (B // N_DEV), B // N_DEV), :]

    return pl.pallas_call(
        body,
        out_shape=jax.ShapeDtypeStruct((B // N_DEV, D), jnp.float32),
        in_specs=[pl.BlockSpec(memory_space=pltpu.VMEM)] * 7,
        out_specs=pl.BlockSpec(memory_space=pltpu.VMEM),
        scratch_shapes=[
            pltpu.VMEM((N_LAYERS, B, D), jnp.bfloat16),
            pltpu.VMEM((N_LAYERS, N_DEV - 1, B, D), jnp.bfloat16),
            pltpu.VMEM((B, D), jnp.float32),
            pltpu.SemaphoreType.DMA((N_LAYERS, N_DEV - 1)),
            pltpu.SemaphoreType.DMA((N_LAYERS, N_DEV - 1)),
        ],
        compiler_params=pltpu.CompilerParams(collective_id=0),
    )(x, Win0, Wout0, Win1, Wout1, Win2, Wout2)
